# baseline (speedup 1.0000x reference)
import functools

import jax
import jax.numpy as jnp
from jax import lax
from jax.experimental import pallas as pl
from jax.experimental.pallas import tpu as pltpu
from jax.experimental.pallas import tpu_sc as plsc

N = 10000
E = 320000
D = 128
HID = 128
NUM_LABELS = 1000
EMB = 32
G = 64

NP = 10240
BLK = 2048
NBLK = NP // BLK
CH = 128
NCH = 80
NCH0 = 160
NCH1 = 0
EPT = CH * NCH
NTILES = 32
EPAD = EPT * NTILES
ROWS_PER_TILE = NP // 16
DEGW = 128


def _make_sc_conv_body(do_gather):
    def body_fn(g_hbm, src_hbm, dst_hbm, zeros_hbm, ones_hbm, out_hbm,
                s0, s1, d0, d1, rows0, rows1, accum,
                isem, gsem0, gsem1, ssem0, ssem1):
        cid = lax.axis_index("c")
        sid = lax.axis_index("s")
        pltpu.sync_copy(zeros_hbm, accum.at[pl.ds(sid * ROWS_PER_TILE, ROWS_PER_TILE)])
        if not do_gather:
            nch = NCH
            cbase = (cid * 16 + sid) * NCH
            pltpu.sync_copy(ones_hbm, rows0)
            pltpu.sync_copy(ones_hbm, rows1)
        else:
            nch = jnp.where(cid == 0, NCH0, NCH1)
            cbase = cid * 16 * NCH0 + sid * nch
        plsc.subcore_barrier()

        def body(i, carry):
            a = cbase + 2 * i
            b = a + 1
            ia_s = pltpu.async_copy(src_hbm.at[a], s0, isem)
            ia_d = pltpu.async_copy(dst_hbm.at[a], d0, isem)
            ib_s = pltpu.async_copy(src_hbm.at[b], s1, isem)
            ib_d = pltpu.async_copy(dst_hbm.at[b], d1, isem)
            ia_s.wait()
            ia_d.wait()
            ib_s.wait()
            ib_d.wait()
            if do_gather:
                ga = pltpu.async_copy(g_hbm.at[s0], rows0, gsem0)
                gb = pltpu.async_copy(g_hbm.at[s1], rows1, gsem1)
                ga.wait()
                gb.wait()
            sa = pltpu.async_copy(rows0, accum.at[d0], ssem0, add=True)
            sb = pltpu.async_copy(rows1, accum.at[d1], ssem1, add=True)
            sa.wait()
            sb.wait()
            return carry

        lax.fori_loop(0, nch // 2, body, 0)
        plsc.subcore_barrier()
        pltpu.sync_copy(
            accum.at[pl.ds(sid * ROWS_PER_TILE, ROWS_PER_TILE)],
            out_hbm.at[cid, pl.ds(sid * ROWS_PER_TILE, ROWS_PER_TILE)],
        )
    return body_fn


@functools.cache
def _sc_conv_kernel(do_gather):
    mesh = plsc.VectorSubcoreMesh(core_axis_name="c", subcore_axis_name="s")
    return pl.kernel(
        _make_sc_conv_body(do_gather),
        out_type=jax.ShapeDtypeStruct((2, NP, HID), jnp.float32),
        mesh=mesh,
        scratch_types=[
            pltpu.VMEM((CH,), jnp.int32),
            pltpu.VMEM((CH,), jnp.int32),
            pltpu.VMEM((CH,), jnp.int32),
            pltpu.VMEM((CH,), jnp.int32),
            pltpu.VMEM((CH, HID), jnp.float32),
            pltpu.VMEM((CH, HID), jnp.float32),
            pltpu.VMEM_SHARED((NP, HID), jnp.float32),
            pltpu.SemaphoreType.DMA,
            pltpu.SemaphoreType.DMA,
            pltpu.SemaphoreType.DMA,
            pltpu.SemaphoreType.DMA,
            pltpu.SemaphoreType.DMA,
        ],
    )


def _sc_conv(g, src_pad3, dst_pad3, zeros_acc, ones_rows, do_gather=True):
    return _sc_conv_kernel(do_gather)(g, src_pad3, dst_pad3, zeros_acc, ones_rows)


def _tc_h1_body(x_ref, drnl_ref, degp_ref, emb_ref, w1x_ref, w1e_ref,
                h1_ref, g1_ref):
    cls_tab = jnp.dot(emb_ref[...], w1e_ref[...],
                      preferred_element_type=jnp.float32)
    dr = drnl_ref[0, 0, :]
    oh = (lax.broadcasted_iota(jnp.int32, (BLK, NUM_LABELS), 1)
          == dr[:, None]).astype(jnp.float32)
    h = (jnp.dot(x_ref[...], w1x_ref[...], preferred_element_type=jnp.float32)
         + jnp.dot(oh, cls_tab, preferred_element_type=jnp.float32))
    deg = 1.0 + degp_ref[0] + degp_ref[1]
    dinv = lax.rsqrt(deg[:, 0:1])
    h1_ref[...] = h
    g1_ref[...] = h * dinv


def _tc_h1(x_pad, drnl2d, degp, emb, w1x, w1e):
    return pl.pallas_call(
        _tc_h1_body,
        grid=(NBLK,),
        in_specs=[
            pl.BlockSpec((BLK, D), lambda i: (i, 0)),
            pl.BlockSpec((1, 1, BLK), lambda i: (i, 0, 0)),
            pl.BlockSpec((2, BLK, DEGW), lambda i: (0, i, 0)),
            pl.BlockSpec((NUM_LABELS, EMB), lambda i: (0, 0)),
            pl.BlockSpec((D, HID), lambda i: (0, 0)),
            pl.BlockSpec((EMB, HID), lambda i: (0, 0)),
        ],
        out_specs=[
            pl.BlockSpec((BLK, HID), lambda i: (i, 0)),
            pl.BlockSpec((BLK, HID), lambda i: (i, 0)),
        ],
        out_shape=[
            jax.ShapeDtypeStruct((NP, HID), jnp.float32),
            jax.ShapeDtypeStruct((NP, HID), jnp.float32),
        ],
    )(x_pad, drnl2d, degp, emb, w1x, w1e)


def _tc_mid_body(sp_ref, h1_ref, degp_ref, w2_ref, b1_ref, h2_ref, g2_ref):
    deg = 1.0 + degp_ref[0] + degp_ref[1]
    dinv = lax.rsqrt(deg[:, 0:1])
    s = sp_ref[0] + sp_ref[1]
    z1 = jax.nn.relu(dinv * s + (dinv * dinv) * h1_ref[...] + b1_ref[...])
    h2 = jnp.dot(z1, w2_ref[...], preferred_element_type=jnp.float32)
    h2_ref[...] = h2
    g2_ref[...] = h2 * dinv


def _tc_mid(sp, h1, degp, w2, b1_2d):
    return pl.pallas_call(
        _tc_mid_body,
        grid=(NBLK,),
        in_specs=[
            pl.BlockSpec((2, BLK, HID), lambda i: (0, i, 0)),
            pl.BlockSpec((BLK, HID), lambda i: (i, 0)),
            pl.BlockSpec((2, BLK, DEGW), lambda i: (0, i, 0)),
            pl.BlockSpec((HID, HID), lambda i: (0, 0)),
            pl.BlockSpec((1, HID), lambda i: (0, 0)),
        ],
        out_specs=[
            pl.BlockSpec((BLK, HID), lambda i: (i, 0)),
            pl.BlockSpec((BLK, HID), lambda i: (i, 0)),
        ],
        out_shape=[
            jax.ShapeDtypeStruct((NP, HID), jnp.float32),
            jax.ShapeDtypeStruct((NP, HID), jnp.float32),
        ],
    )(sp, h1, degp, w2, b1_2d)


def _tc_pool_body(sp_ref, h2_ref, degp_ref, b2_ref, batch_ref,
                  z2_ref, hpool_ref, cnt_ref):
    i = pl.program_id(0)
    deg = 1.0 + degp_ref[0] + degp_ref[1]
    dinv = lax.rsqrt(deg[:, 0:1])
    s = sp_ref[0] + sp_ref[1]
    z2 = jax.nn.relu(dinv * s + (dinv * dinv) * h2_ref[...] + b2_ref[...])
    z2_ref[...] = z2
    bcol = batch_ref[0]

    @pl.when(i == 0)
    def _():
        hpool_ref[...] = jnp.full((G, HID), -jnp.inf, jnp.float32)
        cnt_ref[...] = jnp.zeros((G, HID), jnp.float32)

    def body(g, carry):
        mg = (bcol == g)
        sel = jnp.max(jnp.where(mg, z2, -jnp.inf), axis=0)
        cur = hpool_ref[pl.ds(g, 1), :]
        hpool_ref[pl.ds(g, 1), :] = jnp.maximum(cur, sel[None, :])
        cg = jnp.sum(mg.astype(jnp.float32))
        cnt_ref[pl.ds(g, 1), :] = cnt_ref[pl.ds(g, 1), :] + cg
        return carry

    bmin = jnp.min(bcol)
    bmax = jnp.minimum(jnp.max(bcol), G - 1)
    lax.fori_loop(bmin, bmax + 1, body, 0)


def _tc_pool(sp, h2, degp, b2_2d, batch3d):
    return pl.pallas_call(
        _tc_pool_body,
        grid=(NBLK,),
        in_specs=[
            pl.BlockSpec((2, BLK, HID), lambda i: (0, i, 0)),
            pl.BlockSpec((BLK, HID), lambda i: (i, 0)),
            pl.BlockSpec((2, BLK, DEGW), lambda i: (0, i, 0)),
            pl.BlockSpec((1, HID), lambda i: (0, 0)),
            pl.BlockSpec((1, BLK, 1), lambda i: (i, 0, 0)),
        ],
        out_specs=[
            pl.BlockSpec((BLK, HID), lambda i: (i, 0)),
            pl.BlockSpec((G, HID), lambda i: (0, 0)),
            pl.BlockSpec((G, HID), lambda i: (0, 0)),
        ],
        out_shape=[
            jax.ShapeDtypeStruct((NP, HID), jnp.float32),
            jax.ShapeDtypeStruct((G, HID), jnp.float32),
            jax.ShapeDtypeStruct((G, HID), jnp.float32),
        ],
    )(sp, h2, degp, b2_2d, batch3d)


def _tc_final_body(z2_ref, hpool_ref, cnt_ref, tlu_ref, tlv_ref,
                   wm1_ref, bm1_ref, wm2_ref, bm2_ref, out_ref):
    c = cnt_ref[...]
    ltri = (lax.broadcasted_iota(jnp.int32, (G, G), 0)
            > lax.broadcasted_iota(jnp.int32, (G, G), 1)).astype(jnp.float32)
    ptr_excl = jnp.dot(ltri, c, preferred_element_type=jnp.float32)
    idx_u = (ptr_excl + tlu_ref[...])[:, 0:1]
    idx_v = (ptr_excl + tlv_ref[...])[:, 0:1]
    iota_n = lax.broadcasted_iota(jnp.int32, (G, NP), 1)
    ohu = (iota_n == idx_u.astype(jnp.int32)).astype(jnp.float32)
    ohv = (iota_n == idx_v.astype(jnp.int32)).astype(jnp.float32)
    z2 = z2_ref[...]
    h_u = jnp.dot(ohu, z2, preferred_element_type=jnp.float32)
    h_v = jnp.dot(ohv, z2, preferred_element_type=jnp.float32)
    feats = jnp.concatenate(
        [h_u, h_v, jnp.abs(h_u - h_v), h_u * h_v, hpool_ref[...]], axis=1)
    hidden = jax.nn.relu(
        jnp.dot(feats, wm1_ref[...], preferred_element_type=jnp.float32)
        + bm1_ref[...])
    out_ref[...] = (jnp.dot(hidden, wm2_ref[...],
                            preferred_element_type=jnp.float32) + bm2_ref[...])


def _tc_final(z2, hpool, cnt, tlu_b, tlv_b, wm1, bm1_2d, wm2_pad, bm2_pad):
    return pl.pallas_call(
        _tc_final_body,
        out_shape=jax.ShapeDtypeStruct((G, HID), jnp.float32),
    )(z2, hpool, cnt, tlu_b, tlv_b, wm1, bm1_2d, wm2_pad, bm2_pad)


def kernel(x, edge_index, drnl, batch, target_local, emb,
           W1, b1, W2, b2, Wm1, bm1, Wm2, bm2):
    f32 = jnp.float32
    i32 = jnp.int32

    src = edge_index[0].astype(i32)
    dst = edge_index[1].astype(i32)
    pad_e = jnp.full((EPAD - E,), N, i32)
    src_pad3 = jnp.concatenate([src, pad_e]).reshape(NTILES * NCH, CH)
    dst_pad3 = jnp.concatenate([dst, pad_e]).reshape(NTILES * NCH, CH)

    x_pad = jnp.pad(x.astype(f32), ((0, NP - N), (0, 0)))
    drnl2d = jnp.pad(drnl.astype(i32), (0, NP - N)).reshape(NBLK, 1, BLK)
    batch3d = jnp.pad(batch.astype(i32), (0, NP - N),
                      constant_values=G).reshape(NBLK, BLK, 1)

    ones_g = jnp.ones((8, HID), f32)
    ones_rows = jnp.ones((CH, HID), f32)
    zeros_acc = jnp.zeros((ROWS_PER_TILE, HID), f32)

    w1x = W1[:D].astype(f32)
    w1e = W1[D:].astype(f32)
    b1_2d = b1.reshape(1, HID).astype(f32)
    b2_2d = b2.reshape(1, HID).astype(f32)
    bm1_2d = bm1.reshape(1, HID).astype(f32)
    wm2_pad = jnp.pad(Wm2.astype(f32), ((0, 0), (0, HID - 2)))
    bm2_pad = jnp.pad(bm2.reshape(1, 2).astype(f32), ((0, 0), (0, HID - 2)))
    tl = target_local.astype(f32).reshape(G, 2)
    tlu_b = jnp.broadcast_to(tl[:, 0:1], (G, HID))
    tlv_b = jnp.broadcast_to(tl[:, 1:2], (G, HID))

    degp = _sc_conv(ones_g, dst_pad3, dst_pad3, zeros_acc, ones_rows, do_gather=False)

    h1, g1 = _tc_h1(x_pad, drnl2d, degp, emb.astype(f32), w1x, w1e)
    sp1 = _sc_conv(g1, src_pad3, dst_pad3, zeros_acc, ones_rows)

    h2, g2 = _tc_mid(sp1, h1, degp, W2.astype(f32), b1_2d)
    sp2 = _sc_conv(g2, src_pad3, dst_pad3, zeros_acc, ones_rows)

    z2, hpool, cnt = _tc_pool(sp2, h2, degp, b2_2d, batch3d)
    out = _tc_final(z2, hpool, cnt, tlu_b, tlv_b,
                    Wm1.astype(f32), bm1_2d, wm2_pad, bm2_pad)
    return out[:, :2]


# speedup vs baseline: 1.6475x; 1.6475x over previous
import functools

import jax
import jax.numpy as jnp
from jax import lax
from jax.experimental import pallas as pl
from jax.experimental.pallas import tpu as pltpu
from jax.experimental.pallas import tpu_sc as plsc

N = 10000
E = 320000
D = 128
HID = 128
NUM_LABELS = 1000
EMB = 32
G = 64

NP = 10240
BLK = 2048
NBLK = NP // BLK
CH = 128
NCH = 80
NCH0 = 152
NCH1 = 8
EPT = CH * NCH
NTILES = 32
EPAD = EPT * NTILES
ROWS_PER_TILE = NP // 16
DEGW = 128


def _make_sc_conv_body(do_gather):
    def body_fn(g_hbm, src_hbm, dst_hbm, zeros_hbm, ones_hbm, out_hbm,
                s0, s1, d0, d1, rows0, rows1, accum,
                isem, gsem0, gsem1, ssem0, ssem1):
        cid = lax.axis_index("c")
        sid = lax.axis_index("s")
        pltpu.sync_copy(zeros_hbm, accum.at[pl.ds(sid * ROWS_PER_TILE, ROWS_PER_TILE)])
        if not do_gather:
            nch = NCH
            cbase = (cid * 16 + sid) * NCH
            pltpu.sync_copy(ones_hbm, rows0)
            pltpu.sync_copy(ones_hbm, rows1)
        else:
            nch = jnp.where(cid == 0, NCH0, NCH1)
            cbase = cid * 16 * NCH0 + sid * nch
        plsc.subcore_barrier()

        def body(i, carry):
            a = cbase + 2 * i
            b = a + 1
            ia_s = pltpu.async_copy(src_hbm.at[a], s0, isem)
            ia_d = pltpu.async_copy(dst_hbm.at[a], d0, isem)
            ib_s = pltpu.async_copy(src_hbm.at[b], s1, isem)
            ib_d = pltpu.async_copy(dst_hbm.at[b], d1, isem)
            ia_s.wait()
            ia_d.wait()
            ib_s.wait()
            ib_d.wait()
            if do_gather:
                ga = pltpu.async_copy(g_hbm.at[s0], rows0, gsem0)
                gb = pltpu.async_copy(g_hbm.at[s1], rows1, gsem1)
                ga.wait()
                gb.wait()
            sa = pltpu.async_copy(rows0, accum.at[d0], ssem0, add=True)
            sb = pltpu.async_copy(rows1, accum.at[d1], ssem1, add=True)
            sa.wait()
            sb.wait()
            return carry

        lax.fori_loop(0, nch // 2, body, 0)
        plsc.subcore_barrier()
        pltpu.sync_copy(
            accum.at[pl.ds(sid * ROWS_PER_TILE, ROWS_PER_TILE)],
            out_hbm.at[cid, pl.ds(sid * ROWS_PER_TILE, ROWS_PER_TILE)],
        )
    return body_fn


@functools.cache
def _sc_conv_kernel(do_gather):
    mesh = plsc.VectorSubcoreMesh(core_axis_name="c", subcore_axis_name="s")
    return pl.kernel(
        _make_sc_conv_body(do_gather),
        out_type=jax.ShapeDtypeStruct((2, NP, HID), jnp.float32),
        mesh=mesh,
        scratch_types=[
            pltpu.VMEM((CH,), jnp.int32),
            pltpu.VMEM((CH,), jnp.int32),
            pltpu.VMEM((CH,), jnp.int32),
            pltpu.VMEM((CH,), jnp.int32),
            pltpu.VMEM((CH, HID), jnp.float32),
            pltpu.VMEM((CH, HID), jnp.float32),
            pltpu.VMEM_SHARED((NP, HID), jnp.float32),
            pltpu.SemaphoreType.DMA,
            pltpu.SemaphoreType.DMA,
            pltpu.SemaphoreType.DMA,
            pltpu.SemaphoreType.DMA,
            pltpu.SemaphoreType.DMA,
        ],
    )


def _sc_conv(g, src_pad3, dst_pad3, zeros_acc, ones_rows, do_gather=True):
    return _sc_conv_kernel(do_gather)(g, src_pad3, dst_pad3, zeros_acc, ones_rows)


def _tc_h1_body(x_ref, drnl_ref, degp_ref, emb_ref, w1x_ref, w1e_ref,
                h1_ref, g1_ref):
    cls_tab = jnp.dot(emb_ref[...], w1e_ref[...],
                      preferred_element_type=jnp.float32)
    dr = drnl_ref[0, 0, :]
    oh = (lax.broadcasted_iota(jnp.int32, (BLK, NUM_LABELS), 1)
          == dr[:, None]).astype(jnp.float32)
    h = (jnp.dot(x_ref[...], w1x_ref[...], preferred_element_type=jnp.float32)
         + jnp.dot(oh, cls_tab, preferred_element_type=jnp.float32))
    deg = 1.0 + degp_ref[0] + degp_ref[1]
    dinv = lax.rsqrt(deg[:, 0:1])
    h1_ref[...] = h
    g1_ref[...] = h * dinv


def _tc_h1(x_pad, drnl2d, degp, emb, w1x, w1e):
    return pl.pallas_call(
        _tc_h1_body,
        grid=(NBLK,),
        in_specs=[
            pl.BlockSpec((BLK, D), lambda i: (i, 0)),
            pl.BlockSpec((1, 1, BLK), lambda i: (i, 0, 0)),
            pl.BlockSpec((2, BLK, DEGW), lambda i: (0, i, 0)),
            pl.BlockSpec((NUM_LABELS, EMB), lambda i: (0, 0)),
            pl.BlockSpec((D, HID), lambda i: (0, 0)),
            pl.BlockSpec((EMB, HID), lambda i: (0, 0)),
        ],
        out_specs=[
            pl.BlockSpec((BLK, HID), lambda i: (i, 0)),
            pl.BlockSpec((BLK, HID), lambda i: (i, 0)),
        ],
        out_shape=[
            jax.ShapeDtypeStruct((NP, HID), jnp.float32),
            jax.ShapeDtypeStruct((NP, HID), jnp.float32),
        ],
    )(x_pad, drnl2d, degp, emb, w1x, w1e)


def _tc_mid_body(sp_ref, h1_ref, degp_ref, w2_ref, b1_ref, h2_ref, g2_ref):
    deg = 1.0 + degp_ref[0] + degp_ref[1]
    dinv = lax.rsqrt(deg[:, 0:1])
    s = sp_ref[0] + sp_ref[1]
    z1 = jax.nn.relu(dinv * s + (dinv * dinv) * h1_ref[...] + b1_ref[...])
    h2 = jnp.dot(z1, w2_ref[...], preferred_element_type=jnp.float32)
    h2_ref[...] = h2
    g2_ref[...] = h2 * dinv


def _tc_mid(sp, h1, degp, w2, b1_2d):
    return pl.pallas_call(
        _tc_mid_body,
        grid=(NBLK,),
        in_specs=[
            pl.BlockSpec((2, BLK, HID), lambda i: (0, i, 0)),
            pl.BlockSpec((BLK, HID), lambda i: (i, 0)),
            pl.BlockSpec((2, BLK, DEGW), lambda i: (0, i, 0)),
            pl.BlockSpec((HID, HID), lambda i: (0, 0)),
            pl.BlockSpec((1, HID), lambda i: (0, 0)),
        ],
        out_specs=[
            pl.BlockSpec((BLK, HID), lambda i: (i, 0)),
            pl.BlockSpec((BLK, HID), lambda i: (i, 0)),
        ],
        out_shape=[
            jax.ShapeDtypeStruct((NP, HID), jnp.float32),
            jax.ShapeDtypeStruct((NP, HID), jnp.float32),
        ],
    )(sp, h1, degp, w2, b1_2d)


def _tc_pool_body(sp_ref, h2_ref, degp_ref, b2_ref, batch_ref,
                  z2_ref, hpool_ref, cnt_ref):
    i = pl.program_id(0)
    deg = 1.0 + degp_ref[0] + degp_ref[1]
    dinv = lax.rsqrt(deg[:, 0:1])
    s = sp_ref[0] + sp_ref[1]
    z2 = jax.nn.relu(dinv * s + (dinv * dinv) * h2_ref[...] + b2_ref[...])
    z2_ref[...] = z2
    bcol = batch_ref[0]

    @pl.when(i == 0)
    def _():
        hpool_ref[...] = jnp.full((G, HID), -jnp.inf, jnp.float32)
        cnt_ref[...] = jnp.zeros((G, HID), jnp.float32)

    def body(g, carry):
        mg = (bcol == g)
        sel = jnp.max(jnp.where(mg, z2, -jnp.inf), axis=0)
        cur = hpool_ref[pl.ds(g, 1), :]
        hpool_ref[pl.ds(g, 1), :] = jnp.maximum(cur, sel[None, :])
        cg = jnp.sum(mg.astype(jnp.float32))
        cnt_ref[pl.ds(g, 1), :] = cnt_ref[pl.ds(g, 1), :] + cg
        return carry

    bmin = jnp.min(bcol)
    bmax = jnp.minimum(jnp.max(bcol), G - 1)
    lax.fori_loop(bmin, bmax + 1, body, 0)


def _tc_pool(sp, h2, degp, b2_2d, batch3d):
    return pl.pallas_call(
        _tc_pool_body,
        grid=(NBLK,),
        in_specs=[
            pl.BlockSpec((2, BLK, HID), lambda i: (0, i, 0)),
            pl.BlockSpec((BLK, HID), lambda i: (i, 0)),
            pl.BlockSpec((2, BLK, DEGW), lambda i: (0, i, 0)),
            pl.BlockSpec((1, HID), lambda i: (0, 0)),
            pl.BlockSpec((1, BLK, 1), lambda i: (i, 0, 0)),
        ],
        out_specs=[
            pl.BlockSpec((BLK, HID), lambda i: (i, 0)),
            pl.BlockSpec((G, HID), lambda i: (0, 0)),
            pl.BlockSpec((G, HID), lambda i: (0, 0)),
        ],
        out_shape=[
            jax.ShapeDtypeStruct((NP, HID), jnp.float32),
            jax.ShapeDtypeStruct((G, HID), jnp.float32),
            jax.ShapeDtypeStruct((G, HID), jnp.float32),
        ],
    )(sp, h2, degp, b2_2d, batch3d)


def _tc_final_body(z2_ref, hpool_ref, cnt_ref, tlu_ref, tlv_ref,
                   wm1_ref, bm1_ref, wm2_ref, bm2_ref, out_ref):
    c = cnt_ref[...]
    ltri = (lax.broadcasted_iota(jnp.int32, (G, G), 0)
            > lax.broadcasted_iota(jnp.int32, (G, G), 1)).astype(jnp.float32)
    ptr_excl = jnp.dot(ltri, c, preferred_element_type=jnp.float32)
    idx_u = (ptr_excl + tlu_ref[...])[:, 0:1]
    idx_v = (ptr_excl + tlv_ref[...])[:, 0:1]
    iota_n = lax.broadcasted_iota(jnp.int32, (G, NP), 1)
    ohu = (iota_n == idx_u.astype(jnp.int32)).astype(jnp.float32)
    ohv = (iota_n == idx_v.astype(jnp.int32)).astype(jnp.float32)
    z2 = z2_ref[...]
    h_u = jnp.dot(ohu, z2, preferred_element_type=jnp.float32)
    h_v = jnp.dot(ohv, z2, preferred_element_type=jnp.float32)
    feats = jnp.concatenate(
        [h_u, h_v, jnp.abs(h_u - h_v), h_u * h_v, hpool_ref[...]], axis=1)
    hidden = jax.nn.relu(
        jnp.dot(feats, wm1_ref[...], preferred_element_type=jnp.float32)
        + bm1_ref[...])
    out_ref[...] = (jnp.dot(hidden, wm2_ref[...],
                            preferred_element_type=jnp.float32) + bm2_ref[...])


def _tc_final(z2, hpool, cnt, tlu_b, tlv_b, wm1, bm1_2d, wm2_pad, bm2_pad):
    return pl.pallas_call(
        _tc_final_body,
        out_shape=jax.ShapeDtypeStruct((G, HID), jnp.float32),
    )(z2, hpool, cnt, tlu_b, tlv_b, wm1, bm1_2d, wm2_pad, bm2_pad)


def kernel(x, edge_index, drnl, batch, target_local, emb,
           W1, b1, W2, b2, Wm1, bm1, Wm2, bm2):
    f32 = jnp.float32
    i32 = jnp.int32

    src = edge_index[0].astype(i32)
    dst = edge_index[1].astype(i32)
    pad_e = jnp.full((EPAD - E,), N, i32)
    src_pad3 = jnp.concatenate([src, pad_e]).reshape(NTILES * NCH, CH)
    dst_pad3 = jnp.concatenate([dst, pad_e]).reshape(NTILES * NCH, CH)

    x_pad = jnp.pad(x.astype(f32), ((0, NP - N), (0, 0)))
    drnl2d = jnp.pad(drnl.astype(i32), (0, NP - N)).reshape(NBLK, 1, BLK)
    batch3d = jnp.pad(batch.astype(i32), (0, NP - N),
                      constant_values=G).reshape(NBLK, BLK, 1)

    ones_g = jnp.ones((8, HID), f32)
    ones_rows = jnp.ones((CH, HID), f32)
    zeros_acc = jnp.zeros((ROWS_PER_TILE, HID), f32)

    w1x = W1[:D].astype(f32)
    w1e = W1[D:].astype(f32)
    b1_2d = b1.reshape(1, HID).astype(f32)
    b2_2d = b2.reshape(1, HID).astype(f32)
    bm1_2d = bm1.reshape(1, HID).astype(f32)
    wm2_pad = jnp.pad(Wm2.astype(f32), ((0, 0), (0, HID - 2)))
    bm2_pad = jnp.pad(bm2.reshape(1, 2).astype(f32), ((0, 0), (0, HID - 2)))
    tl = target_local.astype(f32).reshape(G, 2)
    tlu_b = jnp.broadcast_to(tl[:, 0:1], (G, HID))
    tlv_b = jnp.broadcast_to(tl[:, 1:2], (G, HID))

    degp = _sc_conv(ones_g, dst_pad3, dst_pad3, zeros_acc, ones_rows, do_gather=False)

    h1, g1 = _tc_h1(x_pad, drnl2d, degp, emb.astype(f32), w1x, w1e)
    sp1 = _sc_conv(g1, src_pad3, dst_pad3, zeros_acc, ones_rows)

    h2, g2 = _tc_mid(sp1, h1, degp, W2.astype(f32), b1_2d)
    sp2 = _sc_conv(g2, src_pad3, dst_pad3, zeros_acc, ones_rows)

    z2, hpool, cnt = _tc_pool(sp2, h2, degp, b2_2d, batch3d)
    out = _tc_final(z2, hpool, cnt, tlu_b, tlv_b,
                    Wm1.astype(f32), bm1_2d, wm2_pad, bm2_pad)
    return out[:, :2]


# speedup vs baseline: 1.6482x; 1.0004x over previous
import functools

import jax
import jax.numpy as jnp
from jax import lax
from jax.experimental import pallas as pl
from jax.experimental.pallas import tpu as pltpu
from jax.experimental.pallas import tpu_sc as plsc

N = 10000
E = 320000
D = 128
HID = 128
NUM_LABELS = 1000
EMB = 32
G = 64

NP = 10240
BLK = 2048
NBLK = NP // BLK
CH = 128
NCH = 80
NCH0 = 148
NCH1 = 12
EPT = CH * NCH
NTILES = 32
EPAD = EPT * NTILES
ROWS_PER_TILE = NP // 16
DEGW = 128


def _make_sc_conv_body(do_gather):
    def body_fn(g_hbm, src_hbm, dst_hbm, zeros_hbm, ones_hbm, out_hbm,
                s0, s1, d0, d1, rows0, rows1, accum,
                isem, gsem0, gsem1, ssem0, ssem1):
        cid = lax.axis_index("c")
        sid = lax.axis_index("s")
        pltpu.sync_copy(zeros_hbm, accum.at[pl.ds(sid * ROWS_PER_TILE, ROWS_PER_TILE)])
        if not do_gather:
            nch = NCH
            cbase = (cid * 16 + sid) * NCH
            pltpu.sync_copy(ones_hbm, rows0)
            pltpu.sync_copy(ones_hbm, rows1)
        else:
            nch = jnp.where(cid == 0, NCH0, NCH1)
            cbase = cid * 16 * NCH0 + sid * nch
        plsc.subcore_barrier()

        def body(i, carry):
            a = cbase + 2 * i
            b = a + 1
            ia_s = pltpu.async_copy(src_hbm.at[a], s0, isem)
            ia_d = pltpu.async_copy(dst_hbm.at[a], d0, isem)
            ib_s = pltpu.async_copy(src_hbm.at[b], s1, isem)
            ib_d = pltpu.async_copy(dst_hbm.at[b], d1, isem)
            ia_s.wait()
            ia_d.wait()
            ib_s.wait()
            ib_d.wait()
            if do_gather:
                ga = pltpu.async_copy(g_hbm.at[s0], rows0, gsem0)
                gb = pltpu.async_copy(g_hbm.at[s1], rows1, gsem1)
                ga.wait()
                gb.wait()
            sa = pltpu.async_copy(rows0, accum.at[d0], ssem0, add=True)
            sb = pltpu.async_copy(rows1, accum.at[d1], ssem1, add=True)
            sa.wait()
            sb.wait()
            return carry

        lax.fori_loop(0, nch // 2, body, 0)
        plsc.subcore_barrier()
        pltpu.sync_copy(
            accum.at[pl.ds(sid * ROWS_PER_TILE, ROWS_PER_TILE)],
            out_hbm.at[cid, pl.ds(sid * ROWS_PER_TILE, ROWS_PER_TILE)],
        )
    return body_fn


@functools.cache
def _sc_conv_kernel(do_gather):
    mesh = plsc.VectorSubcoreMesh(core_axis_name="c", subcore_axis_name="s")
    return pl.kernel(
        _make_sc_conv_body(do_gather),
        out_type=jax.ShapeDtypeStruct((2, NP, HID), jnp.float32),
        mesh=mesh,
        scratch_types=[
            pltpu.VMEM((CH,), jnp.int32),
            pltpu.VMEM((CH,), jnp.int32),
            pltpu.VMEM((CH,), jnp.int32),
            pltpu.VMEM((CH,), jnp.int32),
            pltpu.VMEM((CH, HID), jnp.float32),
            pltpu.VMEM((CH, HID), jnp.float32),
            pltpu.VMEM_SHARED((NP, HID), jnp.float32),
            pltpu.SemaphoreType.DMA,
            pltpu.SemaphoreType.DMA,
            pltpu.SemaphoreType.DMA,
            pltpu.SemaphoreType.DMA,
            pltpu.SemaphoreType.DMA,
        ],
    )


def _sc_conv(g, src_pad3, dst_pad3, zeros_acc, ones_rows, do_gather=True):
    return _sc_conv_kernel(do_gather)(g, src_pad3, dst_pad3, zeros_acc, ones_rows)


def _tc_h1_body(x_ref, drnl_ref, degp_ref, emb_ref, w1x_ref, w1e_ref,
                h1_ref, g1_ref):
    cls_tab = jnp.dot(emb_ref[...], w1e_ref[...],
                      preferred_element_type=jnp.float32)
    dr = drnl_ref[0, 0, :]
    oh = (lax.broadcasted_iota(jnp.int32, (BLK, NUM_LABELS), 1)
          == dr[:, None]).astype(jnp.float32)
    h = (jnp.dot(x_ref[...], w1x_ref[...], preferred_element_type=jnp.float32)
         + jnp.dot(oh, cls_tab, preferred_element_type=jnp.float32))
    deg = 1.0 + degp_ref[0] + degp_ref[1]
    dinv = lax.rsqrt(deg[:, 0:1])
    h1_ref[...] = h
    g1_ref[...] = h * dinv


def _tc_h1(x_pad, drnl2d, degp, emb, w1x, w1e):
    return pl.pallas_call(
        _tc_h1_body,
        grid=(NBLK,),
        in_specs=[
            pl.BlockSpec((BLK, D), lambda i: (i, 0)),
            pl.BlockSpec((1, 1, BLK), lambda i: (i, 0, 0)),
            pl.BlockSpec((2, BLK, DEGW), lambda i: (0, i, 0)),
            pl.BlockSpec((NUM_LABELS, EMB), lambda i: (0, 0)),
            pl.BlockSpec((D, HID), lambda i: (0, 0)),
            pl.BlockSpec((EMB, HID), lambda i: (0, 0)),
        ],
        out_specs=[
            pl.BlockSpec((BLK, HID), lambda i: (i, 0)),
            pl.BlockSpec((BLK, HID), lambda i: (i, 0)),
        ],
        out_shape=[
            jax.ShapeDtypeStruct((NP, HID), jnp.float32),
            jax.ShapeDtypeStruct((NP, HID), jnp.float32),
        ],
    )(x_pad, drnl2d, degp, emb, w1x, w1e)


def _tc_mid_body(sp_ref, h1_ref, degp_ref, w2_ref, b1_ref, h2_ref, g2_ref):
    deg = 1.0 + degp_ref[0] + degp_ref[1]
    dinv = lax.rsqrt(deg[:, 0:1])
    s = sp_ref[0] + sp_ref[1]
    z1 = jax.nn.relu(dinv * s + (dinv * dinv) * h1_ref[...] + b1_ref[...])
    h2 = jnp.dot(z1, w2_ref[...], preferred_element_type=jnp.float32)
    h2_ref[...] = h2
    g2_ref[...] = h2 * dinv


def _tc_mid(sp, h1, degp, w2, b1_2d):
    return pl.pallas_call(
        _tc_mid_body,
        grid=(NBLK,),
        in_specs=[
            pl.BlockSpec((2, BLK, HID), lambda i: (0, i, 0)),
            pl.BlockSpec((BLK, HID), lambda i: (i, 0)),
            pl.BlockSpec((2, BLK, DEGW), lambda i: (0, i, 0)),
            pl.BlockSpec((HID, HID), lambda i: (0, 0)),
            pl.BlockSpec((1, HID), lambda i: (0, 0)),
        ],
        out_specs=[
            pl.BlockSpec((BLK, HID), lambda i: (i, 0)),
            pl.BlockSpec((BLK, HID), lambda i: (i, 0)),
        ],
        out_shape=[
            jax.ShapeDtypeStruct((NP, HID), jnp.float32),
            jax.ShapeDtypeStruct((NP, HID), jnp.float32),
        ],
    )(sp, h1, degp, w2, b1_2d)


def _tc_pool_body(sp_ref, h2_ref, degp_ref, b2_ref, batch_ref,
                  z2_ref, hpool_ref, cnt_ref):
    i = pl.program_id(0)
    deg = 1.0 + degp_ref[0] + degp_ref[1]
    dinv = lax.rsqrt(deg[:, 0:1])
    s = sp_ref[0] + sp_ref[1]
    z2 = jax.nn.relu(dinv * s + (dinv * dinv) * h2_ref[...] + b2_ref[...])
    z2_ref[...] = z2
    bcol = batch_ref[0]

    @pl.when(i == 0)
    def _():
        hpool_ref[...] = jnp.full((G, HID), -jnp.inf, jnp.float32)
        cnt_ref[...] = jnp.zeros((G, HID), jnp.float32)

    def body(g, carry):
        mg = (bcol == g)
        sel = jnp.max(jnp.where(mg, z2, -jnp.inf), axis=0)
        cur = hpool_ref[pl.ds(g, 1), :]
        hpool_ref[pl.ds(g, 1), :] = jnp.maximum(cur, sel[None, :])
        cg = jnp.sum(mg.astype(jnp.float32))
        cnt_ref[pl.ds(g, 1), :] = cnt_ref[pl.ds(g, 1), :] + cg
        return carry

    bmin = jnp.min(bcol)
    bmax = jnp.minimum(jnp.max(bcol), G - 1)
    lax.fori_loop(bmin, bmax + 1, body, 0)


def _tc_pool(sp, h2, degp, b2_2d, batch3d):
    return pl.pallas_call(
        _tc_pool_body,
        grid=(NBLK,),
        in_specs=[
            pl.BlockSpec((2, BLK, HID), lambda i: (0, i, 0)),
            pl.BlockSpec((BLK, HID), lambda i: (i, 0)),
            pl.BlockSpec((2, BLK, DEGW), lambda i: (0, i, 0)),
            pl.BlockSpec((1, HID), lambda i: (0, 0)),
            pl.BlockSpec((1, BLK, 1), lambda i: (i, 0, 0)),
        ],
        out_specs=[
            pl.BlockSpec((BLK, HID), lambda i: (i, 0)),
            pl.BlockSpec((G, HID), lambda i: (0, 0)),
            pl.BlockSpec((G, HID), lambda i: (0, 0)),
        ],
        out_shape=[
            jax.ShapeDtypeStruct((NP, HID), jnp.float32),
            jax.ShapeDtypeStruct((G, HID), jnp.float32),
            jax.ShapeDtypeStruct((G, HID), jnp.float32),
        ],
    )(sp, h2, degp, b2_2d, batch3d)


def _tc_final_body(z2_ref, hpool_ref, cnt_ref, tlu_ref, tlv_ref,
                   wm1_ref, bm1_ref, wm2_ref, bm2_ref, out_ref):
    c = cnt_ref[...]
    ltri = (lax.broadcasted_iota(jnp.int32, (G, G), 0)
            > lax.broadcasted_iota(jnp.int32, (G, G), 1)).astype(jnp.float32)
    ptr_excl = jnp.dot(ltri, c, preferred_element_type=jnp.float32)
    idx_u = (ptr_excl + tlu_ref[...])[:, 0:1]
    idx_v = (ptr_excl + tlv_ref[...])[:, 0:1]
    iota_n = lax.broadcasted_iota(jnp.int32, (G, NP), 1)
    ohu = (iota_n == idx_u.astype(jnp.int32)).astype(jnp.float32)
    ohv = (iota_n == idx_v.astype(jnp.int32)).astype(jnp.float32)
    z2 = z2_ref[...]
    h_u = jnp.dot(ohu, z2, preferred_element_type=jnp.float32)
    h_v = jnp.dot(ohv, z2, preferred_element_type=jnp.float32)
    feats = jnp.concatenate(
        [h_u, h_v, jnp.abs(h_u - h_v), h_u * h_v, hpool_ref[...]], axis=1)
    hidden = jax.nn.relu(
        jnp.dot(feats, wm1_ref[...], preferred_element_type=jnp.float32)
        + bm1_ref[...])
    out_ref[...] = (jnp.dot(hidden, wm2_ref[...],
                            preferred_element_type=jnp.float32) + bm2_ref[...])


def _tc_final(z2, hpool, cnt, tlu_b, tlv_b, wm1, bm1_2d, wm2_pad, bm2_pad):
    return pl.pallas_call(
        _tc_final_body,
        out_shape=jax.ShapeDtypeStruct((G, HID), jnp.float32),
    )(z2, hpool, cnt, tlu_b, tlv_b, wm1, bm1_2d, wm2_pad, bm2_pad)


def kernel(x, edge_index, drnl, batch, target_local, emb,
           W1, b1, W2, b2, Wm1, bm1, Wm2, bm2):
    f32 = jnp.float32
    i32 = jnp.int32

    src = edge_index[0].astype(i32)
    dst = edge_index[1].astype(i32)
    pad_e = jnp.full((EPAD - E,), N, i32)
    src_pad3 = jnp.concatenate([src, pad_e]).reshape(NTILES * NCH, CH)
    dst_pad3 = jnp.concatenate([dst, pad_e]).reshape(NTILES * NCH, CH)

    x_pad = jnp.pad(x.astype(f32), ((0, NP - N), (0, 0)))
    drnl2d = jnp.pad(drnl.astype(i32), (0, NP - N)).reshape(NBLK, 1, BLK)
    batch3d = jnp.pad(batch.astype(i32), (0, NP - N),
                      constant_values=G).reshape(NBLK, BLK, 1)

    ones_g = jnp.ones((8, HID), f32)
    ones_rows = jnp.ones((CH, HID), f32)
    zeros_acc = jnp.zeros((ROWS_PER_TILE, HID), f32)

    w1x = W1[:D].astype(f32)
    w1e = W1[D:].astype(f32)
    b1_2d = b1.reshape(1, HID).astype(f32)
    b2_2d = b2.reshape(1, HID).astype(f32)
    bm1_2d = bm1.reshape(1, HID).astype(f32)
    wm2_pad = jnp.pad(Wm2.astype(f32), ((0, 0), (0, HID - 2)))
    bm2_pad = jnp.pad(bm2.reshape(1, 2).astype(f32), ((0, 0), (0, HID - 2)))
    tl = target_local.astype(f32).reshape(G, 2)
    tlu_b = jnp.broadcast_to(tl[:, 0:1], (G, HID))
    tlv_b = jnp.broadcast_to(tl[:, 1:2], (G, HID))

    degp = _sc_conv(ones_g, dst_pad3, dst_pad3, zeros_acc, ones_rows, do_gather=False)

    h1, g1 = _tc_h1(x_pad, drnl2d, degp, emb.astype(f32), w1x, w1e)
    sp1 = _sc_conv(g1, src_pad3, dst_pad3, zeros_acc, ones_rows)

    h2, g2 = _tc_mid(sp1, h1, degp, W2.astype(f32), b1_2d)
    sp2 = _sc_conv(g2, src_pad3, dst_pad3, zeros_acc, ones_rows)

    z2, hpool, cnt = _tc_pool(sp2, h2, degp, b2_2d, batch3d)
    out = _tc_final(z2, hpool, cnt, tlu_b, tlv_b,
                    Wm1.astype(f32), bm1_2d, wm2_pad, bm2_pad)
    return out[:, :2]
